# baseline (speedup 1.0000x reference)
import jax
import jax.numpy as jnp
import numpy as np
from jax.experimental import pallas as pl
from jax.experimental.pallas import tpu as pltpu

_OUT = 32
_PROJ = _OUT * (_OUT + 1) // 2
_FLAT = _OUT * _OUT


def _sym_perm() -> np.ndarray:
    rows, cols = np.triu_indices(_OUT)
    m = np.zeros((_OUT, _OUT), dtype=np.int32)
    m[rows, cols] = np.arange(_PROJ, dtype=np.int32)
    m[cols, rows] = np.arange(_PROJ, dtype=np.int32)
    return m.reshape(-1)


_G = _sym_perm()


def _proj_kernel(x_ref, w_ref, b_ref, o_ref):
    y = (
        jnp.dot(x_ref[...], w_ref[...], preferred_element_type=jnp.float32)
        + b_ref[...]
    )
    o_ref[...] = y.reshape(y.shape[0], 8, 128)


def kernel(node_feats, W, b):
    n, d = node_feats.shape
    w2 = W[_G].T.astype(jnp.float32)
    b2 = b[_G][None, :].astype(jnp.float32)

    bn = 2000
    if n % bn != 0:
        bn = next(s for s in (1000, 500, 200, 100, 50, 25, 8, 1) if n % s == 0)
    grid = n // bn

    out = pl.pallas_call(
        _proj_kernel,
        grid=(grid,),
        in_specs=[
            pl.BlockSpec((bn, d), lambda i: (i, 0)),
            pl.BlockSpec((d, _FLAT), lambda i: (0, 0)),
            pl.BlockSpec((1, _FLAT), lambda i: (0, 0)),
        ],
        out_specs=pl.BlockSpec((bn, 8, 128), lambda i: (i, 0, 0)),
        out_shape=jax.ShapeDtypeStruct((n, 8, 128), jnp.float32),
        compiler_params=pltpu.CompilerParams(
            dimension_semantics=("parallel",)
        ),
    )(node_feats, w2, b2)
    return out.reshape(n, _OUT, _OUT)


# speedup vs baseline: 3.1173x; 3.1173x over previous
import jax
import jax.numpy as jnp
import numpy as np
from jax.experimental import pallas as pl
from jax.experimental.pallas import tpu as pltpu

_OUT = 32
_PROJ = _OUT * (_OUT + 1) // 2
_FLAT = _OUT * _OUT


def _sym_perm() -> np.ndarray:
    rows, cols = np.triu_indices(_OUT)
    m = np.zeros((_OUT, _OUT), dtype=np.int32)
    m[rows, cols] = np.arange(_PROJ, dtype=np.int32)
    m[cols, rows] = np.arange(_PROJ, dtype=np.int32)
    return m.reshape(-1)


_G = _sym_perm()


def _proj_kernel(w_ref, x_ref, b_ref, o_ref):
    o_ref[...] = (
        jax.lax.dot_general(
            w_ref[...],
            x_ref[...],
            (((1,), (1,)), ((), ())),
            preferred_element_type=jnp.float32,
        )
        + b_ref[...]
    )


def kernel(node_feats, W, b):
    n, d = node_feats.shape
    w2 = W[_G].astype(jnp.float32)
    b2 = b[_G][:, None].astype(jnp.float32)

    bn = 2048
    grid = (n + bn - 1) // bn

    out_t = pl.pallas_call(
        _proj_kernel,
        grid=(grid,),
        in_specs=[
            pl.BlockSpec((_FLAT, d), lambda i: (0, 0)),
            pl.BlockSpec((bn, d), lambda i: (i, 0)),
            pl.BlockSpec((_FLAT, 1), lambda i: (0, 0)),
        ],
        out_specs=pl.BlockSpec((_FLAT, bn), lambda i: (0, i)),
        out_shape=jax.ShapeDtypeStruct((_FLAT, n), jnp.float32),
        compiler_params=pltpu.CompilerParams(
            dimension_semantics=("parallel",)
        ),
    )(w2, node_feats, b2)
    return out_t.reshape(_OUT, _OUT, n).transpose(2, 0, 1)


# speedup vs baseline: 3.1400x; 1.0073x over previous
import jax
import jax.numpy as jnp
import numpy as np
from jax.experimental import pallas as pl
from jax.experimental.pallas import tpu as pltpu

_OUT = 32
_PROJ = _OUT * (_OUT + 1) // 2
_FLAT = _OUT * _OUT


def _sym_perm() -> np.ndarray:
    rows, cols = np.triu_indices(_OUT)
    m = np.zeros((_OUT, _OUT), dtype=np.int32)
    m[rows, cols] = np.arange(_PROJ, dtype=np.int32)
    m[cols, rows] = np.arange(_PROJ, dtype=np.int32)
    return m.reshape(-1)


_G = _sym_perm()


def _proj_kernel(w_ref, x_ref, b_ref, o_ref):
    o_ref[...] = (
        jax.lax.dot_general(
            w_ref[...],
            x_ref[...],
            (((1,), (1,)), ((), ())),
            preferred_element_type=jnp.float32,
        )
        + b_ref[...]
    )


def kernel(node_feats, W, b):
    n, d = node_feats.shape
    w2 = W[_G].astype(jnp.float32)
    b2 = b[_G][:, None].astype(jnp.float32)

    bn = 4096
    grid = (n + bn - 1) // bn

    out_t = pl.pallas_call(
        _proj_kernel,
        grid=(grid,),
        in_specs=[
            pl.BlockSpec((_FLAT, d), lambda i: (0, 0)),
            pl.BlockSpec((bn, d), lambda i: (i, 0)),
            pl.BlockSpec((_FLAT, 1), lambda i: (0, 0)),
        ],
        out_specs=pl.BlockSpec((_FLAT, bn), lambda i: (0, i)),
        out_shape=jax.ShapeDtypeStruct((_FLAT, n), jnp.float32),
        compiler_params=pltpu.CompilerParams(
            dimension_semantics=("parallel",)
        ),
    )(w2, node_feats, b2)
    return out_t.reshape(_OUT, _OUT, n).transpose(2, 0, 1)
